# Initial kernel scaffold; baseline (speedup 1.0000x reference)
import jax
import jax.numpy as jnp
from jax.experimental import pallas as pl


def kernel(x, xp, alpha, beta):
    raise NotImplementedError("write your pallas kernel here")


# speedup vs baseline: 2.2908x; 2.2908x over previous
import functools

import jax
import jax.numpy as jnp
from jax import lax
from jax.experimental import pallas as pl
from jax.experimental.pallas import tpu as pltpu
from jax.experimental.pallas import tpu_sc as plsc

_B = 16384
_NW = 32
_CHUNK = _B // _NW
_L = 16


def _sc_body(x_hbm, xp_hbm, alpha_hbm, beta_hbm, out_hbm,
             xv, xpv, av, bv, ev, outv):
    wid = lax.axis_index("s") * 2 + lax.axis_index("c")
    base = wid * _CHUNK
    pltpu.sync_copy(x_hbm.at[pl.ds(2 * base, 2 * _CHUNK)], xv)
    pltpu.sync_copy(xp_hbm.at[pl.ds(2 * base, 2 * _CHUNK)], xpv)
    pltpu.sync_copy(alpha_hbm, av)
    pltpu.sync_copy(beta_hbm, bv)

    beta = bv[...]
    dgrid = lax.iota(jnp.int32, _L).astype(jnp.float32)
    ev[...] = beta * jnp.exp(-beta * dgrid)

    def step(j, carry):
        r2 = 2 * (j * _L + lax.iota(jnp.int32, _L))
        x0 = plsc.load_gather(xv, [r2])
        x1 = plsc.load_gather(xv, [r2 + 1])
        xp0 = plsc.load_gather(xpv, [r2])
        xp1 = plsc.load_gather(xpv, [r2 + 1])
        dt = jnp.abs(x0 - xp0)
        a_ = plsc.load_gather(av, [xp1 * 8 + x1])
        e_ = plsc.load_gather(ev, [dt])
        outv[pl.ds(j * _L, _L)] = a_ * e_
        return carry

    lax.fori_loop(0, _CHUNK // _L, step, 0)
    pltpu.sync_copy(outv, out_hbm.at[pl.ds(base, _CHUNK)])


@functools.partial(
    pl.kernel,
    out_type=jax.ShapeDtypeStruct((_B,), jnp.float32),
    mesh=plsc.VectorSubcoreMesh(core_axis_name="c", subcore_axis_name="s"),
    compiler_params=pltpu.CompilerParams(needs_layout_passes=False),
    scratch_types=[
        pltpu.VMEM((2 * _CHUNK,), jnp.int32),
        pltpu.VMEM((2 * _CHUNK,), jnp.int32),
        pltpu.VMEM((64,), jnp.float32),
        pltpu.VMEM((_L,), jnp.float32),
        pltpu.VMEM((_L,), jnp.float32),
        pltpu.VMEM((_CHUNK,), jnp.float32),
    ],
)
def _sc_kernel(x_hbm, xp_hbm, alpha_hbm, beta_hbm, out_hbm, *scratch):
    _sc_body(x_hbm, xp_hbm, alpha_hbm, beta_hbm, out_hbm, *scratch)


def kernel(x, xp, alpha, beta):
    beta16 = jnp.broadcast_to(beta.astype(jnp.float32), (_L,))
    return _sc_kernel(x.reshape(-1), xp.reshape(-1), alpha.reshape(-1), beta16)
